# baseline (speedup 1.0000x reference)
import functools

import jax
import jax.numpy as jnp
from jax import lax
from jax.experimental import pallas as pl
from jax.experimental.pallas import tpu as pltpu
from jax.experimental.pallas import tpu_sc as plsc

B = 16384
D = 64
LANES = 16
NUM_CORES = 2
NUM_SUBCORES = 16
NW = NUM_CORES * NUM_SUBCORES
BPW = B // NW
IDX_CHUNK = 128
NCHUNK = BPW // IDX_CHUNK


def _sc_partials(z, labels3, centers):
    mesh = plsc.VectorSubcoreMesh(core_axis_name="c", subcore_axis_name="s")

    @functools.partial(
        pl.kernel,
        out_type=jax.ShapeDtypeStruct((NW, LANES), jnp.float32),
        mesh=mesh,
        scratch_types=[
            pltpu.VMEM((NCHUNK, IDX_CHUNK), jnp.int32),
            pltpu.VMEM((BPW, D), jnp.float32),
            pltpu.VMEM((BPW, D), jnp.float32),
            pltpu.VMEM((LANES,), jnp.float32),
            pltpu.SemaphoreType.DMA,
        ],
        compiler_params=pltpu.CompilerParams(use_tc_tiling_on_sc=False),
    )
    def k(z_hbm, labels_hbm, centers_hbm, out_hbm, idx_v, c_v, z_v, acc_v, sem):
        wid = lax.axis_index("s") * NUM_CORES + lax.axis_index("c")
        base = wid * BPW

        pltpu.sync_copy(labels_hbm.at[wid], idx_v)
        gathers = [
            pltpu.async_copy(
                centers_hbm.at[idx_v.at[j]],
                c_v.at[pl.ds(j * IDX_CHUNK, IDX_CHUNK)],
                sem,
            )
            for j in range(NCHUNK)
        ]
        pltpu.sync_copy(z_hbm.at[pl.ds(base, BPW)], z_v)
        for g in gathers:
            g.wait()

        def body(i, accs):
            out = []
            for j in range(D // LANES):
                dz = z_v[i, pl.ds(j * LANES, LANES)] - c_v[i, pl.ds(j * LANES, LANES)]
                out.append(accs[j] + dz * dz)
            return tuple(out)

        zero = jnp.zeros((LANES,), jnp.float32)
        accs = lax.fori_loop(0, BPW, body, (zero,) * (D // LANES))
        acc_v[...] = accs[0] + accs[1] + accs[2] + accs[3]
        pltpu.sync_copy(acc_v, out_hbm.at[wid])

    return k(z, labels3, centers)


def _reduce_partials(partials):
    def body(p_ref, o_ref):
        o_ref[0, 0] = jnp.sum(p_ref[...]) * (1.0 / B)

    out = pl.pallas_call(
        body,
        out_shape=jax.ShapeDtypeStruct((1, 1), jnp.float32),
        out_specs=pl.BlockSpec(memory_space=pltpu.SMEM),
    )(partials)
    return out[0, 0]


def kernel(z, labels, centers):
    labels3 = labels.astype(jnp.int32).reshape(NW, NCHUNK, IDX_CHUNK)
    partials = _sc_partials(z, labels3, centers)
    return _reduce_partials(partials)
